# baseline (speedup 1.0000x reference)
import jax
import jax.numpy as jnp
import numpy as np
from jax.experimental import pallas as pl
from jax.experimental.pallas import tpu as pltpu

H_P, W_P = 14, 14
N_TOK = H_P * W_P
EMBED_DIM = 768
NUM_HEADS = 12
DEPTH = 12
HIDDEN_DIM = 3072
R_C, R_F = 4, 8
TAU = 1.0
EPS = 1e-6
ALPHA, BETA, GAMMA = 1.0, 0.5, 0.5
RHO = 0.5
TOK_TEMP = 0.1
B, T = 8, 16
K_TOP = max(1, int(RHO * T * N_TOK))


def _interp_matrix(t_in, t_out):
    src = (np.arange(t_out, dtype=np.float64) + 0.5) * (t_in / float(t_out)) - 0.5
    src = np.clip(src, 0.0, t_in - 1.0)
    lo = np.floor(src).astype(np.int64)
    hi = np.minimum(lo + 1, t_in - 1)
    w = (src - lo).astype(np.float32)
    m = np.zeros((t_out, t_in), dtype=np.float32)
    m[np.arange(t_out), lo] += 1.0 - w
    m[np.arange(t_out), hi] += w
    return m


M2_NP = _interp_matrix(T - 1, T)
M4_NP = _interp_matrix(T - 3, T)


def _region_one_hot(coords, centers):
    d = jnp.sqrt(jnp.maximum(
        ((coords[:, None, :] - centers[None, :, :]) ** 2).sum(-1), 0.0))
    rid = jnp.argmin(d, axis=1)
    return (rid[None, :] == jnp.arange(centers.shape[0])[:, None]).astype(
        jnp.float32)


def _entropy_kernel(x_ref, oh_ref,
                    ent1_ref, ent2_ref, ent4_ref, hc_ref, hf_ref,
                    hist_ref):
    t = pl.program_id(1)
    x = x_ref[0, 0]
    m = jnp.max(x, axis=1, keepdims=True)
    e = jnp.exp((x - m) * (1.0 / TAU))
    z = jnp.sum(e, axis=1, keepdims=True)
    p = e / z

    @pl.when(t == 0)
    def _init():
        hist_ref[3] = jnp.zeros((N_TOK, EMBED_DIM), jnp.float32)

    s_prev1 = hist_ref[jax.lax.rem(t + 3, 4)]
    s_t = s_prev1 + p

    q1 = s_t - s_prev1
    ent1_ref[0, t, :] = -jnp.sum(q1 * jnp.log(q1 + EPS), axis=1)

    @pl.when(t >= 1)
    def _l2():
        s2 = hist_ref[jax.lax.rem(t + 2, 4)]
        q = (s_t - s2) * 0.5
        ent2_ref[0, t, :] = -jnp.sum(q * jnp.log(q + EPS), axis=1)

    @pl.when(t == 0)
    def _l2z():
        ent2_ref[0, 0, :] = jnp.zeros((N_TOK,), jnp.float32)

    @pl.when(t >= 3)
    def _l4():
        s4 = hist_ref[jax.lax.rem(t, 4)]
        q = (s_t - s4) * 0.25
        ent4_ref[0, t, :] = -jnp.sum(q * jnp.log(q + EPS), axis=1)

    @pl.when(t < 3)
    def _l4z():
        ent4_ref[0, t, :] = jnp.zeros((N_TOK,), jnp.float32)

    hist_ref[jax.lax.rem(t, 4)] = s_t

    oh = oh_ref[...]
    cnt = jnp.sum(oh, axis=1, keepdims=True)
    p_sum = jnp.dot(oh, p, preferred_element_type=jnp.float32)
    p_reg = p_sum / (cnt + EPS)
    ent_r = -jnp.sum(p_reg * jnp.log(p_reg + EPS), axis=1)
    hc_ref[0, t, :] = ent_r[:R_C]
    hf_ref[0, t, :] = ent_r[R_C:]


def _entropy_pass(x, oh):
    n, c = N_TOK, EMBED_DIM
    return pl.pallas_call(
        _entropy_kernel,
        grid=(B, T),
        in_specs=[
            pl.BlockSpec((1, 1, n, c), lambda b, t: (b, t, 0, 0)),
            pl.BlockSpec((R_C + R_F, n), lambda b, t: (0, 0)),
        ],
        out_specs=[
            pl.BlockSpec((1, T, n), lambda b, t: (b, 0, 0)),
            pl.BlockSpec((1, T, n), lambda b, t: (b, 0, 0)),
            pl.BlockSpec((1, T, n), lambda b, t: (b, 0, 0)),
            pl.BlockSpec((1, T, R_C), lambda b, t: (b, 0, 0)),
            pl.BlockSpec((1, T, R_F), lambda b, t: (b, 0, 0)),
        ],
        out_shape=[
            jax.ShapeDtypeStruct((B, T, n), jnp.float32),
            jax.ShapeDtypeStruct((B, T, n), jnp.float32),
            jax.ShapeDtypeStruct((B, T, n), jnp.float32),
            jax.ShapeDtypeStruct((B, T, R_C), jnp.float32),
            jax.ShapeDtypeStruct((B, T, R_F), jnp.float32),
        ],
        scratch_shapes=[pltpu.VMEM((4, n, c), jnp.float32)],
    )(x, oh)


def _normalize(h):
    mn = jnp.min(h)
    mx = jnp.max(h)
    return (h - mn) / (mx - mn + EPS)


def _kth_largest(score, k):
    hi0 = jnp.max(score) + 1.0
    lo0 = jnp.zeros((), jnp.float32)

    def body(_, carry):
        lo, hi = carry
        mid = 0.5 * (lo + hi)
        cnt = jnp.sum((score >= mid).astype(jnp.float32))
        ge = cnt >= float(k)
        return jnp.where(ge, mid, lo), jnp.where(ge, hi, mid)

    lo, _ = jax.lax.fori_loop(0, 50, body, (lo0, hi0))
    return lo


def _mask_kernel(ent1_ref, ent2_ref, ent4_ref, hc_ref, hf_ref,
                 oh_ref,
                 ghead_ref, gch_ref, gblock_ref, m2_ref, m4_ref,
                 mask_ref, headw_ref, chw_ref, blockw_ref, st_ref, last_ref):
    m2 = m2_ref[...]
    m4 = m4_ref[...]
    oh_c = oh_ref[:R_C, :]
    oh_f = oh_ref[R_C:, :]

    total = jnp.zeros((), jnp.float32)
    for b in range(B):
        e1 = ent1_ref[b]
        e2 = ent2_ref[b][1:T, :]
        e4 = ent4_ref[b][3:T, :]
        i2 = jnp.dot(m2, e2, preferred_element_type=jnp.float32, precision=jax.lax.Precision.HIGHEST)
        i4 = jnp.dot(m4, e4, preferred_element_type=jnp.float32, precision=jax.lax.Precision.HIGHEST)
        ht = (e1 + i2 + i4) * (1.0 / 3.0)
        ht_n = _normalize(ht)
        hc_n = _normalize(hc_ref[b])
        hf_n = _normalize(hf_ref[b])
        hc_tok = jnp.dot(hc_n, oh_c, preferred_element_type=jnp.float32, precision=jax.lax.Precision.HIGHEST)
        hf_tok = jnp.dot(hf_n, oh_f, preferred_element_type=jnp.float32, precision=jax.lax.Precision.HIGHEST)
        score = ALPHA * ht_n + BETA * hc_tok + GAMMA * hf_tok
        kth = _kth_largest(score, K_TOP)
        mask = jax.nn.sigmoid((score - kth) * (1.0 / TOK_TEMP))
        mask_ref[b] = mask
        total = total + jnp.sum(mask)

    sparsity_token = 1.0 - total / float(B * T * N_TOK)
    head_w = jax.nn.sigmoid(ghead_ref[...])
    ch_w = jax.nn.sigmoid(gch_ref[...])
    block_w = jax.nn.sigmoid(gblock_ref[...])
    headw_ref[...] = head_w
    chw_ref[...] = ch_w
    blockw_ref[...] = block_w
    l_ast = (sparsity_token + (1.0 - jnp.mean(head_w))
             + (1.0 - jnp.mean(ch_w)) + (1.0 - jnp.mean(block_w)))
    st_ref[...] = jnp.reshape(sparsity_token, (1, 1))
    last_ref[...] = jnp.reshape(l_ast, (1, 1))


def _mask_pass(ent1, ent2, ent4, hc, hf, oh,
               g_head, g_ch, g_block2d):
    return pl.pallas_call(
        _mask_kernel,
        out_shape=[
            jax.ShapeDtypeStruct((B, T, N_TOK), jnp.float32),
            jax.ShapeDtypeStruct((DEPTH, NUM_HEADS), jnp.float32),
            jax.ShapeDtypeStruct((DEPTH, HIDDEN_DIM), jnp.float32),
            jax.ShapeDtypeStruct((1, DEPTH), jnp.float32),
            jax.ShapeDtypeStruct((1, 1), jnp.float32),
            jax.ShapeDtypeStruct((1, 1), jnp.float32),
        ],
    )(ent1, ent2, ent4, hc, hf, oh, g_head, g_ch, g_block2d,
      jnp.asarray(M2_NP), jnp.asarray(M4_NP))


def kernel(token_feat, centers_coarse, centers_fine, g_head, g_ch, g_block,
           patch_coords):
    oh = jnp.concatenate([
        _region_one_hot(patch_coords, centers_coarse),
        _region_one_hot(patch_coords, centers_fine),
    ], axis=0)
    ent1, ent2, ent4, hc, hf = _entropy_pass(token_feat, oh)
    mask, head_w, ch_w, block_w2, st, last = _mask_pass(
        ent1, ent2, ent4, hc, hf, oh,
        g_head, g_ch, g_block.reshape(1, DEPTH))
    return (mask, head_w, ch_w, block_w2.reshape(DEPTH),
            st.reshape(()), last.reshape(()))


# speedup vs baseline: 1.3674x; 1.3674x over previous
import jax
import jax.numpy as jnp
import numpy as np
from jax.experimental import pallas as pl
from jax.experimental.pallas import tpu as pltpu

H_P, W_P = 14, 14
N_TOK = H_P * W_P
EMBED_DIM = 768
NUM_HEADS = 12
DEPTH = 12
HIDDEN_DIM = 3072
R_C, R_F = 4, 8
TAU = 1.0
EPS = 1e-6
ALPHA, BETA, GAMMA = 1.0, 0.5, 0.5
RHO = 0.5
TOK_TEMP = 0.1
B, T = 8, 16
K_TOP = max(1, int(RHO * T * N_TOK))


def _interp_matrix(t_in, t_out):
    src = (np.arange(t_out, dtype=np.float64) + 0.5) * (t_in / float(t_out)) - 0.5
    src = np.clip(src, 0.0, t_in - 1.0)
    lo = np.floor(src).astype(np.int64)
    hi = np.minimum(lo + 1, t_in - 1)
    w = (src - lo).astype(np.float32)
    m = np.zeros((t_out, t_in), dtype=np.float32)
    m[np.arange(t_out), lo] += 1.0 - w
    m[np.arange(t_out), hi] += w
    return m


M2_NP = _interp_matrix(T - 1, T)
M4_NP = _interp_matrix(T - 3, T)


def _region_one_hot(coords, centers):
    d = jnp.sqrt(jnp.maximum(
        ((coords[:, None, :] - centers[None, :, :]) ** 2).sum(-1), 0.0))
    rid = jnp.argmin(d, axis=1)
    return (rid[None, :] == jnp.arange(centers.shape[0])[:, None]).astype(
        jnp.float32)


def _entropy_kernel(x_ref, oh_ref,
                    ent1_ref, ent2_ref, ent4_ref, hc_ref, hf_ref):
    oh = oh_ref[...]
    cnt = jnp.sum(oh, axis=1, keepdims=True)

    def _h(q):
        return -jnp.sum(q * jnp.log(q + EPS), axis=1)

    s_hist = []
    for t in range(T):
        x = x_ref[0, t]
        m = jnp.max(x, axis=1, keepdims=True)
        e = jnp.exp((x - m) * (1.0 / TAU))
        z = jnp.sum(e, axis=1, keepdims=True)
        p = e / z

        s_t = p if t == 0 else s_hist[t - 1] + p
        q1 = p if t == 0 else s_t - s_hist[t - 1]
        ent1_ref[0, t, :] = _h(q1)

        if t == 0:
            ent2_ref[0, 0, :] = jnp.zeros((N_TOK,), jnp.float32)
        else:
            q2 = s_t * 0.5 if t == 1 else (s_t - s_hist[t - 2]) * 0.5
            ent2_ref[0, t, :] = _h(q2)

        if t < 3:
            ent4_ref[0, t, :] = jnp.zeros((N_TOK,), jnp.float32)
        else:
            q4 = s_t * 0.25 if t == 3 else (s_t - s_hist[t - 4]) * 0.25
            ent4_ref[0, t, :] = _h(q4)

        s_hist.append(s_t)

        p_sum = jnp.dot(oh, p, preferred_element_type=jnp.float32)
        p_reg = p_sum / (cnt + EPS)
        ent_r = _h(p_reg)
        hc_ref[0, t, :] = ent_r[:R_C]
        hf_ref[0, t, :] = ent_r[R_C:]


def _entropy_pass(x, oh):
    n, c = N_TOK, EMBED_DIM
    return pl.pallas_call(
        _entropy_kernel,
        grid=(B,),
        in_specs=[
            pl.BlockSpec((1, T, n, c), lambda b: (b, 0, 0, 0)),
            pl.BlockSpec((R_C + R_F, n), lambda b: (0, 0)),
        ],
        out_specs=[
            pl.BlockSpec((1, T, n), lambda b: (b, 0, 0)),
            pl.BlockSpec((1, T, n), lambda b: (b, 0, 0)),
            pl.BlockSpec((1, T, n), lambda b: (b, 0, 0)),
            pl.BlockSpec((1, T, R_C), lambda b: (b, 0, 0)),
            pl.BlockSpec((1, T, R_F), lambda b: (b, 0, 0)),
        ],
        out_shape=[
            jax.ShapeDtypeStruct((B, T, n), jnp.float32),
            jax.ShapeDtypeStruct((B, T, n), jnp.float32),
            jax.ShapeDtypeStruct((B, T, n), jnp.float32),
            jax.ShapeDtypeStruct((B, T, R_C), jnp.float32),
            jax.ShapeDtypeStruct((B, T, R_F), jnp.float32),
        ],
    )(x, oh)


def _normalize(h):
    mn = jnp.min(h)
    mx = jnp.max(h)
    return (h - mn) / (mx - mn + EPS)


def _kth_largest(score, k):
    hi0 = jnp.max(score) + 1.0
    lo0 = jnp.zeros((), jnp.float32)

    def body(_, carry):
        lo, hi = carry
        mid = 0.5 * (lo + hi)
        cnt = jnp.sum((score >= mid).astype(jnp.float32))
        ge = cnt >= float(k)
        return jnp.where(ge, mid, lo), jnp.where(ge, hi, mid)

    lo, _ = jax.lax.fori_loop(0, 50, body, (lo0, hi0))
    return lo


def _mask_kernel(ent1_ref, ent2_ref, ent4_ref, hc_ref, hf_ref,
                 oh_ref,
                 ghead_ref, gch_ref, gblock_ref, m2_ref, m4_ref,
                 mask_ref, headw_ref, chw_ref, blockw_ref, st_ref, last_ref):
    m2 = m2_ref[...]
    m4 = m4_ref[...]
    oh_c = oh_ref[:R_C, :]
    oh_f = oh_ref[R_C:, :]

    total = jnp.zeros((), jnp.float32)
    for b in range(B):
        e1 = ent1_ref[b]
        e2 = ent2_ref[b][1:T, :]
        e4 = ent4_ref[b][3:T, :]
        i2 = jnp.dot(m2, e2, preferred_element_type=jnp.float32, precision=jax.lax.Precision.HIGHEST)
        i4 = jnp.dot(m4, e4, preferred_element_type=jnp.float32, precision=jax.lax.Precision.HIGHEST)
        ht = (e1 + i2 + i4) * (1.0 / 3.0)
        ht_n = _normalize(ht)
        hc_n = _normalize(hc_ref[b])
        hf_n = _normalize(hf_ref[b])
        hc_tok = jnp.dot(hc_n, oh_c, preferred_element_type=jnp.float32, precision=jax.lax.Precision.HIGHEST)
        hf_tok = jnp.dot(hf_n, oh_f, preferred_element_type=jnp.float32, precision=jax.lax.Precision.HIGHEST)
        score = ALPHA * ht_n + BETA * hc_tok + GAMMA * hf_tok
        kth = _kth_largest(score, K_TOP)
        mask = jax.nn.sigmoid((score - kth) * (1.0 / TOK_TEMP))
        mask_ref[b] = mask
        total = total + jnp.sum(mask)

    sparsity_token = 1.0 - total / float(B * T * N_TOK)
    head_w = jax.nn.sigmoid(ghead_ref[...])
    ch_w = jax.nn.sigmoid(gch_ref[...])
    block_w = jax.nn.sigmoid(gblock_ref[...])
    headw_ref[...] = head_w
    chw_ref[...] = ch_w
    blockw_ref[...] = block_w
    l_ast = (sparsity_token + (1.0 - jnp.mean(head_w))
             + (1.0 - jnp.mean(ch_w)) + (1.0 - jnp.mean(block_w)))
    st_ref[...] = jnp.reshape(sparsity_token, (1, 1))
    last_ref[...] = jnp.reshape(l_ast, (1, 1))


def _mask_pass(ent1, ent2, ent4, hc, hf, oh,
               g_head, g_ch, g_block2d):
    return pl.pallas_call(
        _mask_kernel,
        out_shape=[
            jax.ShapeDtypeStruct((B, T, N_TOK), jnp.float32),
            jax.ShapeDtypeStruct((DEPTH, NUM_HEADS), jnp.float32),
            jax.ShapeDtypeStruct((DEPTH, HIDDEN_DIM), jnp.float32),
            jax.ShapeDtypeStruct((1, DEPTH), jnp.float32),
            jax.ShapeDtypeStruct((1, 1), jnp.float32),
            jax.ShapeDtypeStruct((1, 1), jnp.float32),
        ],
    )(ent1, ent2, ent4, hc, hf, oh, g_head, g_ch, g_block2d,
      jnp.asarray(M2_NP), jnp.asarray(M4_NP))


def kernel(token_feat, centers_coarse, centers_fine, g_head, g_ch, g_block,
           patch_coords):
    oh = jnp.concatenate([
        _region_one_hot(patch_coords, centers_coarse),
        _region_one_hot(patch_coords, centers_fine),
    ], axis=0)
    ent1, ent2, ent4, hc, hf = _entropy_pass(token_feat, oh)
    mask, head_w, ch_w, block_w2, st, last = _mask_pass(
        ent1, ent2, ent4, hc, hf, oh,
        g_head, g_ch, g_block.reshape(1, DEPTH))
    return (mask, head_w, ch_w, block_w2.reshape(DEPTH),
            st.reshape(()), last.reshape(()))


# speedup vs baseline: 1.3717x; 1.0032x over previous
import jax
import jax.numpy as jnp
import numpy as np
from jax.experimental import pallas as pl
from jax.experimental.pallas import tpu as pltpu

H_P, W_P = 14, 14
N_TOK = H_P * W_P
EMBED_DIM = 768
NUM_HEADS = 12
DEPTH = 12
HIDDEN_DIM = 3072
R_C, R_F = 4, 8
TAU = 1.0
EPS = 1e-6
ALPHA, BETA, GAMMA = 1.0, 0.5, 0.5
RHO = 0.5
TOK_TEMP = 0.1
B, T = 8, 16
K_TOP = max(1, int(RHO * T * N_TOK))


def _interp_matrix(t_in, t_out):
    src = (np.arange(t_out, dtype=np.float64) + 0.5) * (t_in / float(t_out)) - 0.5
    src = np.clip(src, 0.0, t_in - 1.0)
    lo = np.floor(src).astype(np.int64)
    hi = np.minimum(lo + 1, t_in - 1)
    w = (src - lo).astype(np.float32)
    m = np.zeros((t_out, t_in), dtype=np.float32)
    m[np.arange(t_out), lo] += 1.0 - w
    m[np.arange(t_out), hi] += w
    return m


M2_NP = _interp_matrix(T - 1, T)
M4_NP = _interp_matrix(T - 3, T)


def _region_one_hot(coords, centers):
    d = jnp.sqrt(jnp.maximum(
        ((coords[:, None, :] - centers[None, :, :]) ** 2).sum(-1), 0.0))
    rid = jnp.argmin(d, axis=1)
    return (rid[None, :] == jnp.arange(centers.shape[0])[:, None]).astype(
        jnp.float32)


def _normalize(h):
    mn = jnp.min(h)
    mx = jnp.max(h)
    return (h - mn) / (mx - mn + EPS)


def _kth_largest(score, k):
    hi0 = jnp.max(score) + 1.0
    lo0 = jnp.zeros((), jnp.float32)

    def body(_, carry):
        lo, hi = carry
        mid = 0.5 * (lo + hi)
        cnt = jnp.sum((score >= mid).astype(jnp.float32))
        ge = cnt >= float(k)
        return jnp.where(ge, mid, lo), jnp.where(ge, hi, mid)

    lo, _ = jax.lax.fori_loop(0, 50, body, (lo0, hi0))
    return lo


def _fused_kernel(x_ref, oh_ref, m2_ref, m4_ref,
                  ghead_ref, gch_ref, gblock_ref,
                  mask_ref, headw_ref, chw_ref, blockw_ref, st_ref, last_ref,
                  e1_scr, e2_scr, e4_scr, hr_scr, tot_scr):
    b = pl.program_id(0)
    oh = oh_ref[...]
    cnt = jnp.sum(oh, axis=1, keepdims=True)

    def _h(q):
        return -jnp.sum(q * jnp.log(q + EPS), axis=1)

    s_hist = []
    for t in range(T):
        x = x_ref[0, t]
        m = jnp.max(x, axis=1, keepdims=True)
        e = jnp.exp(x - m)
        z = jnp.sum(e, axis=1, keepdims=True)
        p = e / z

        s_t = p if t == 0 else s_hist[t - 1] + p
        q1 = p if t == 0 else s_t - s_hist[t - 1]
        e1_scr[t, :] = _h(q1)

        if t == 0:
            e2_scr[0, :] = jnp.zeros((N_TOK,), jnp.float32)
        else:
            q2 = s_t * 0.5 if t == 1 else (s_t - s_hist[t - 2]) * 0.5
            e2_scr[t, :] = _h(q2)

        if t >= 3:
            q4 = s_t * 0.25 if t == 3 else (s_t - s_hist[t - 4]) * 0.25
            e4_scr[t, :] = _h(q4)

        s_hist.append(s_t)

        p_sum = jnp.dot(oh, p, preferred_element_type=jnp.float32)
        p_reg = p_sum / (cnt + EPS)
        hr_scr[t, :] = _h(p_reg)

    hp = jax.lax.Precision.HIGHEST
    e1 = e1_scr[...]
    e2 = e2_scr[1:T, :]
    e4 = e4_scr[3:T, :]
    i2 = jnp.dot(m2_ref[...], e2, preferred_element_type=jnp.float32,
                 precision=hp)
    i4 = jnp.dot(m4_ref[...], e4, preferred_element_type=jnp.float32,
                 precision=hp)
    ht = (e1 + i2 + i4) * (1.0 / 3.0)
    ht_n = _normalize(ht)
    hr = hr_scr[...]
    hc_n = _normalize(hr[:, :R_C])
    hf_n = _normalize(hr[:, R_C:])
    hc_tok = jnp.dot(hc_n, oh[:R_C, :], preferred_element_type=jnp.float32,
                     precision=hp)
    hf_tok = jnp.dot(hf_n, oh[R_C:, :], preferred_element_type=jnp.float32,
                     precision=hp)
    score = ALPHA * ht_n + BETA * hc_tok + GAMMA * hf_tok
    kth = _kth_largest(score, K_TOP)
    mask = jax.nn.sigmoid((score - kth) * (1.0 / TOK_TEMP))
    mask_ref[0] = mask

    prev = jnp.where(b == 0, 0.0, tot_scr[...][0, 0])
    total = prev + jnp.sum(mask)
    tot_scr[...] = jnp.reshape(total, (1, 1))

    head_w = jax.nn.sigmoid(ghead_ref[...])
    ch_w = jax.nn.sigmoid(gch_ref[...])
    block_w = jax.nn.sigmoid(gblock_ref[...])
    headw_ref[...] = head_w
    chw_ref[...] = ch_w
    blockw_ref[...] = block_w
    sparsity_token = 1.0 - total / float(B * T * N_TOK)
    l_ast = (sparsity_token + (1.0 - jnp.mean(head_w))
             + (1.0 - jnp.mean(ch_w)) + (1.0 - jnp.mean(block_w)))
    st_ref[...] = jnp.reshape(sparsity_token, (1, 1))
    last_ref[...] = jnp.reshape(l_ast, (1, 1))


def kernel(token_feat, centers_coarse, centers_fine, g_head, g_ch, g_block,
           patch_coords):
    oh = jnp.concatenate([
        _region_one_hot(patch_coords, centers_coarse),
        _region_one_hot(patch_coords, centers_fine),
    ], axis=0)

    n, c = N_TOK, EMBED_DIM
    const = lambda b: (0, 0)
    mask, head_w, ch_w, block_w2, st, last = pl.pallas_call(
        _fused_kernel,
        grid=(B,),
        in_specs=[
            pl.BlockSpec((1, T, n, c), lambda b: (b, 0, 0, 0)),
            pl.BlockSpec((R_C + R_F, n), const),
            pl.BlockSpec((T, T - 1), const),
            pl.BlockSpec((T, T - 3), const),
            pl.BlockSpec((DEPTH, NUM_HEADS), const),
            pl.BlockSpec((DEPTH, HIDDEN_DIM), const),
            pl.BlockSpec((1, DEPTH), const),
        ],
        out_specs=[
            pl.BlockSpec((1, T, n), lambda b: (b, 0, 0)),
            pl.BlockSpec((DEPTH, NUM_HEADS), const),
            pl.BlockSpec((DEPTH, HIDDEN_DIM), const),
            pl.BlockSpec((1, DEPTH), const),
            pl.BlockSpec((1, 1), const),
            pl.BlockSpec((1, 1), const),
        ],
        out_shape=[
            jax.ShapeDtypeStruct((B, T, n), jnp.float32),
            jax.ShapeDtypeStruct((DEPTH, NUM_HEADS), jnp.float32),
            jax.ShapeDtypeStruct((DEPTH, HIDDEN_DIM), jnp.float32),
            jax.ShapeDtypeStruct((1, DEPTH), jnp.float32),
            jax.ShapeDtypeStruct((1, 1), jnp.float32),
            jax.ShapeDtypeStruct((1, 1), jnp.float32),
        ],
        scratch_shapes=[
            pltpu.VMEM((T, n), jnp.float32),
            pltpu.VMEM((T, n), jnp.float32),
            pltpu.VMEM((T, n), jnp.float32),
            pltpu.VMEM((T, R_C + R_F), jnp.float32),
            pltpu.VMEM((1, 1), jnp.float32),
        ],
    )(token_feat, oh, jnp.asarray(M2_NP), jnp.asarray(M4_NP),
      g_head, g_ch, g_block.reshape(1, DEPTH))
    return (mask, head_w, ch_w, block_w2.reshape(DEPTH),
            st.reshape(()), last.reshape(()))


# speedup vs baseline: 1.4977x; 1.0919x over previous
import jax
import jax.numpy as jnp
import numpy as np
from jax.experimental import pallas as pl
from jax.experimental.pallas import tpu as pltpu

H_P, W_P = 14, 14
N_TOK = H_P * W_P
EMBED_DIM = 768
NUM_HEADS = 12
DEPTH = 12
HIDDEN_DIM = 3072
R_C, R_F = 4, 8
TAU = 1.0
EPS = 1e-6
ALPHA, BETA, GAMMA = 1.0, 0.5, 0.5
RHO = 0.5
TOK_TEMP = 0.1
B, T = 8, 16
K_TOP = max(1, int(RHO * T * N_TOK))


def _interp_coeffs(t_in, t_out):
    src = ((np.arange(t_out, dtype=np.float32) + np.float32(0.5))
           * np.float32(t_in / float(t_out)) - np.float32(0.5))
    src = np.clip(src, np.float32(0.0), np.float32(t_in - 1.0))
    lo = np.floor(src).astype(np.int32)
    hi = np.minimum(lo + 1, t_in - 1)
    w = (src - lo.astype(np.float32)).astype(np.float32)
    return lo, hi, w


I2_COEF = _interp_coeffs(T - 1, T)
I4_COEF = _interp_coeffs(T - 3, T)


def _interp_rows(e, coef):
    lo, hi, w = coef
    rows = []
    for t in range(T):
        w1 = float(np.float32(1.0) - w[t])
        rows.append(e[int(lo[t])] * w1 + e[int(hi[t])] * float(w[t]))
    return jnp.stack(rows, axis=0)


def _region_one_hot(coords, centers):
    d = jnp.sqrt(jnp.maximum(
        ((coords[:, None, :] - centers[None, :, :]) ** 2).sum(-1), 0.0))
    rid = jnp.argmin(d, axis=1)
    return (rid[None, :] == jnp.arange(centers.shape[0])[:, None]).astype(
        jnp.float32)


def _normalize(h):
    mn = jnp.min(h)
    mx = jnp.max(h)
    return (h - mn) / (mx - mn + EPS)


def _kth_largest(score, k):
    hi0 = jnp.max(score) + 1.0
    lo0 = jnp.zeros((), jnp.float32)

    def body(_, carry):
        lo, hi = carry
        mid = 0.5 * (lo + hi)
        cnt = jnp.sum((score >= mid).astype(jnp.float32))
        ge = cnt >= float(k)
        return jnp.where(ge, mid, lo), jnp.where(ge, hi, mid)

    lo, _ = jax.lax.fori_loop(0, 32, body, (lo0, hi0))
    return lo


def _fused_kernel(x_ref, oh_ref,
                  ghead_ref, gch_ref, gblock_ref,
                  mask_ref, headw_ref, chw_ref, blockw_ref, st_ref, last_ref,
                  e1_scr, e2_scr, e4_scr, hr_scr, tot_scr):
    b = pl.program_id(0)
    oh = oh_ref[...]
    cnt = jnp.sum(oh, axis=1, keepdims=True)

    def _h(q):
        return -jnp.sum(q * jnp.log(q + EPS), axis=1)

    s_hist = []
    for t in range(T):
        x = x_ref[0, t]
        m = jnp.max(x, axis=1, keepdims=True)
        e = jnp.exp(x - m)
        z = jnp.sum(e, axis=1, keepdims=True)
        p = e / z

        s_t = p if t == 0 else s_hist[t - 1] + p
        q1 = p if t == 0 else s_t - s_hist[t - 1]
        e1_scr[t, :] = _h(q1)

        if t == 0:
            e2_scr[0, :] = jnp.zeros((N_TOK,), jnp.float32)
        else:
            q2 = s_t * 0.5 if t == 1 else (s_t - s_hist[t - 2]) * 0.5
            e2_scr[t, :] = _h(q2)

        if t >= 3:
            q4 = s_t * 0.25 if t == 3 else (s_t - s_hist[t - 4]) * 0.25
            e4_scr[t, :] = _h(q4)

        s_hist.append(s_t)

        p_sum = jnp.dot(oh, p, preferred_element_type=jnp.float32)
        p_reg = p_sum / (cnt + EPS)
        hr_scr[t, :] = _h(p_reg)

    e1 = e1_scr[...]
    e2 = e2_scr[1:T, :]
    e4 = e4_scr[3:T, :]
    i2 = _interp_rows(e2, I2_COEF)
    i4 = _interp_rows(e4, I4_COEF)
    ht = (e1 + i2 + i4) * (1.0 / 3.0)
    ht_n = _normalize(ht)
    hr = hr_scr[...]
    hc_n = _normalize(hr[:, :R_C])
    hf_n = _normalize(hr[:, R_C:])
    hc_tok = sum(hc_n[:, r:r + 1] * oh[r:r + 1, :] for r in range(R_C))
    hf_tok = sum(hf_n[:, r:r + 1] * oh[R_C + r:R_C + r + 1, :]
                 for r in range(R_F))
    score = ALPHA * ht_n + BETA * hc_tok + GAMMA * hf_tok
    kth = _kth_largest(score, K_TOP)
    mask = jax.nn.sigmoid((score - kth) * (1.0 / TOK_TEMP))
    mask_ref[0] = mask

    prev = jnp.where(b == 0, 0.0, tot_scr[...][0, 0])
    total = prev + jnp.sum(mask)
    tot_scr[...] = jnp.reshape(total, (1, 1))

    head_w = jax.nn.sigmoid(ghead_ref[...])
    ch_w = jax.nn.sigmoid(gch_ref[...])
    block_w = jax.nn.sigmoid(gblock_ref[...])
    headw_ref[...] = head_w
    chw_ref[...] = ch_w
    blockw_ref[...] = block_w
    sparsity_token = 1.0 - total / float(B * T * N_TOK)
    l_ast = (sparsity_token + (1.0 - jnp.mean(head_w))
             + (1.0 - jnp.mean(ch_w)) + (1.0 - jnp.mean(block_w)))
    st_ref[...] = jnp.reshape(sparsity_token, (1, 1))
    last_ref[...] = jnp.reshape(l_ast, (1, 1))


def kernel(token_feat, centers_coarse, centers_fine, g_head, g_ch, g_block,
           patch_coords):
    oh = jnp.concatenate([
        _region_one_hot(patch_coords, centers_coarse),
        _region_one_hot(patch_coords, centers_fine),
    ], axis=0)

    n, c = N_TOK, EMBED_DIM
    const = lambda b: (0, 0)
    mask, head_w, ch_w, block_w2, st, last = pl.pallas_call(
        _fused_kernel,
        grid=(B,),
        in_specs=[
            pl.BlockSpec((1, T, n, c), lambda b: (b, 0, 0, 0)),
            pl.BlockSpec((R_C + R_F, n), const),
            pl.BlockSpec((DEPTH, NUM_HEADS), const),
            pl.BlockSpec((DEPTH, HIDDEN_DIM), const),
            pl.BlockSpec((1, DEPTH), const),
        ],
        out_specs=[
            pl.BlockSpec((1, T, n), lambda b: (b, 0, 0)),
            pl.BlockSpec((DEPTH, NUM_HEADS), const),
            pl.BlockSpec((DEPTH, HIDDEN_DIM), const),
            pl.BlockSpec((1, DEPTH), const),
            pl.BlockSpec((1, 1), const),
            pl.BlockSpec((1, 1), const),
        ],
        out_shape=[
            jax.ShapeDtypeStruct((B, T, n), jnp.float32),
            jax.ShapeDtypeStruct((DEPTH, NUM_HEADS), jnp.float32),
            jax.ShapeDtypeStruct((DEPTH, HIDDEN_DIM), jnp.float32),
            jax.ShapeDtypeStruct((1, DEPTH), jnp.float32),
            jax.ShapeDtypeStruct((1, 1), jnp.float32),
            jax.ShapeDtypeStruct((1, 1), jnp.float32),
        ],
        scratch_shapes=[
            pltpu.VMEM((T, n), jnp.float32),
            pltpu.VMEM((T, n), jnp.float32),
            pltpu.VMEM((T, n), jnp.float32),
            pltpu.VMEM((T, R_C + R_F), jnp.float32),
            pltpu.VMEM((1, 1), jnp.float32),
        ],
    )(token_feat, oh, g_head, g_ch, g_block.reshape(1, DEPTH))
    return (mask, head_w, ch_w, block_w2.reshape(DEPTH),
            st.reshape(()), last.reshape(()))
